# Initial kernel scaffold; baseline (speedup 1.0000x reference)
import jax
import jax.numpy as jnp
from jax.experimental import pallas as pl


def kernel(node_feat, src, dst, neg, W1, b1, W2, b2, W3, b3, g1, be1, g2, be2, fc1_W, fc1_b, fc2_W, fc2_b):
    raise NotImplementedError("write your pallas kernel here")


# speedup vs baseline: 3.1634x; 3.1634x over previous
import jax
import jax.numpy as jnp
from jax import lax
from jax.experimental import pallas as pl
from jax.experimental.pallas import tpu as pltpu
from jax.experimental.pallas import tpu_sc as plsc

N = 10000
D = 128
HD = D // 2
E = 320000
NC = 2
NS = 16
NW = NC * NS
EPW = E // NW
NP = 10112
RPT = NP // NS
CH = 80
NCHUNK = EPW // CH
DEGW = 64
TCC = 400

_MESH = plsc.VectorSubcoreMesh(core_axis_name="c", subcore_axis_name="s",
                               num_cores=NC, num_subcores=NS)

_f32 = jnp.float32
_i32 = jnp.int32


def _wid():
    return lax.axis_index("c") * NS + lax.axis_index("s")


def _zero_fill(ref, rows, width):
    per_row = width // 16

    def body(i, _):
        r = i // per_row
        c = i % per_row
        ref[r, pl.ds(c * 16, 16)] = jnp.zeros((16,), _f32)
        return 0

    lax.fori_loop(0, rows * per_row, body, 0)


def _deg_body(dst_hbm, out_hbm, idx_v, ones_v, zb_v, shared, sem):
    cid = lax.axis_index("c")
    sid = lax.axis_index("s")
    base = _wid() * EPW

    def fill_ones(i, _):
        r = i // (DEGW // 16)
        c = i % (DEGW // 16)
        ones_v[r, pl.ds(c * 16, 16)] = jnp.ones((16,), _f32)
        return 0
    lax.fori_loop(0, CH * (DEGW // 16), fill_ones, 0)
    _zero_fill(zb_v, RPT, DEGW)
    pltpu.sync_copy(zb_v, shared.at[pl.ds(sid * RPT, RPT), :])
    plsc.subcore_barrier()

    def chunk(i, _):
        pltpu.sync_copy(dst_hbm.at[pl.ds(base + i * CH, CH)], idx_v)
        pltpu.sync_copy(ones_v, shared.at[idx_v], add=True)
        return 0
    lax.fori_loop(0, NCHUNK, chunk, 0)

    plsc.subcore_barrier()
    pltpu.sync_copy(shared.at[pl.ds(sid * RPT, RPT), :], zb_v)
    pltpu.sync_copy(zb_v, out_hbm.at[cid, pl.ds(sid * RPT, RPT), :])


_deg_kernel = pl.kernel(
    _deg_body,
    out_type=jax.ShapeDtypeStruct((NC, NP, DEGW), _f32),
    mesh=_MESH,
    compiler_params=pltpu.CompilerParams(use_tc_tiling_on_sc=False,
                                         needs_layout_passes=False),
    scratch_types=[
        pltpu.VMEM((CH,), _i32),
        pltpu.VMEM((CH, DEGW), _f32),
        pltpu.VMEM((RPT, DEGW), _f32),
        pltpu.VMEM_SHARED((NP, DEGW), _f32),
        pltpu.SemaphoreType.DMA,
    ],
)


def _scatter_body(hsa_hbm, hsb_hbm, src_hbm, dst_hbm, out_hbm,
                  idxs_v, idxd_v, rows_v, zb_v, shared, sem):
    cid = lax.axis_index("c")
    sid = lax.axis_index("s")
    base = _wid() * EPW

    _zero_fill(zb_v, RPT, HD)
    for h, hs_hbm in enumerate((hsa_hbm, hsb_hbm)):
        pltpu.sync_copy(zb_v, shared.at[pl.ds(sid * RPT, RPT), :])
        plsc.subcore_barrier()

        def chunk(i, _):
            pltpu.sync_copy(src_hbm.at[pl.ds(base + i * CH, CH)], idxs_v)
            pltpu.sync_copy(dst_hbm.at[pl.ds(base + i * CH, CH)], idxd_v)
            pltpu.async_copy(hs_hbm.at[idxs_v], rows_v, sem).wait()
            pltpu.sync_copy(rows_v, shared.at[idxd_v], add=True)
            return 0
        lax.fori_loop(0, NCHUNK, chunk, 0)

        plsc.subcore_barrier()
        pltpu.sync_copy(shared.at[pl.ds(sid * RPT, RPT), :], zb_v)
        pltpu.sync_copy(zb_v, out_hbm.at[h, cid, pl.ds(sid * RPT, RPT), :])
        _zero_fill(zb_v, RPT, HD)


_scatter_kernel = pl.kernel(
    _scatter_body,
    out_type=jax.ShapeDtypeStruct((2, NC, NP, HD), _f32),
    mesh=_MESH,
    compiler_params=pltpu.CompilerParams(use_tc_tiling_on_sc=False,
                                         needs_layout_passes=False),
    scratch_types=[
        pltpu.VMEM((CH,), _i32),
        pltpu.VMEM((CH,), _i32),
        pltpu.VMEM((CH, HD), _f32),
        pltpu.VMEM((RPT, HD), _f32),
        pltpu.VMEM_SHARED((NP, HD), _f32),
        pltpu.SemaphoreType.DMA,
    ],
)


def _decode_body(p_hbm, q_hbm, src_hbm, dst_hbm, neg_hbm, wb_hbm,
                 pos_hbm, negout_hbm,
                 idxs_v, idxd_v, idxn_v, bp_v, bqd_v, bqn_v, wv_v,
                 op_v, on_v, sem):
    base = _wid() * EPW
    pltpu.sync_copy(wb_hbm, wv_v)
    lanes = lax.broadcasted_iota(_i32, (16,), 0)

    def chunk(i, _):
        pltpu.sync_copy(src_hbm.at[pl.ds(base + i * CH, CH)], idxs_v)
        pltpu.sync_copy(dst_hbm.at[pl.ds(base + i * CH, CH)], idxd_v)
        pltpu.sync_copy(neg_hbm.at[pl.ds(base + i * CH, CH)], idxn_v)
        pltpu.async_copy(p_hbm.at[idxs_v], bp_v, sem).wait()
        pltpu.async_copy(q_hbm.at[idxd_v], bqd_v, sem).wait()
        pltpu.async_copy(q_hbm.at[idxn_v], bqn_v, sem).wait()
        c_const = wv_v[pl.ds(D, 16)]
        for g in range(CH // 16):
            rows = lanes + g * 16

            def col(j, carry):
                ap, an = carry
                cols = jnp.full((16,), j, _i32)
                p = plsc.load_gather(bp_v, [rows, cols])
                qd = plsc.load_gather(bqd_v, [rows, cols])
                qn = plsc.load_gather(bqn_v, [rows, cols])
                w = plsc.load_gather(wv_v, [cols])
                ap = ap + jnp.maximum(p + qd, 0.0) * w
                an = an + jnp.maximum(p + qn, 0.0) * w
                return ap, an

            z16 = jnp.zeros((16,), _f32)
            ap, an = lax.fori_loop(0, D, col, (z16, z16))
            sp = 1.0 / (1.0 + jnp.exp(-(ap + c_const)))
            sn = 1.0 / (1.0 + jnp.exp(-(an + c_const)))
            op_v[pl.ds(i * CH + g * 16, 16)] = sp
            on_v[pl.ds(i * CH + g * 16, 16)] = sn
        return 0
    lax.fori_loop(0, NCHUNK, chunk, 0)

    pltpu.sync_copy(op_v, pos_hbm.at[pl.ds(base, EPW)])
    pltpu.sync_copy(on_v, negout_hbm.at[pl.ds(base, EPW)])


_decode_kernel = pl.kernel(
    _decode_body,
    out_type=(jax.ShapeDtypeStruct((E,), _f32),
              jax.ShapeDtypeStruct((E,), _f32)),
    mesh=_MESH,
    compiler_params=pltpu.CompilerParams(needs_layout_passes=False),
    scratch_types=[
        pltpu.VMEM((CH,), _i32),
        pltpu.VMEM((CH,), _i32),
        pltpu.VMEM((CH,), _i32),
        pltpu.VMEM((CH, D), _f32),
        pltpu.VMEM((CH, D), _f32),
        pltpu.VMEM((CH, D), _f32),
        pltpu.VMEM((D + 16,), _f32),
        pltpu.VMEM((EPW,), _f32),
        pltpu.VMEM((EPW,), _f32),
        pltpu.SemaphoreType.DMA,
    ],
)


_HI = jax.lax.Precision.HIGHEST


def _mm(a, b):
    return jnp.dot(a, b, precision=_HI, preferred_element_type=_f32)


def _tc_prep_body(nf_ref, w1_ref, degp_ref, hsa_ref, hsb_ref, dinv_ref):
    w1 = w1_ref[...]
    for i in range(N // TCC):
        sl = pl.ds(i * TCC, TCC)
        deg = degp_ref[0, sl, 0:1] + degp_ref[1, sl, 0:1] + 1.0
        dinv = lax.rsqrt(deg)
        dinv_ref[sl] = dinv
        h = _mm(nf_ref[sl, :], w1) * dinv
        hsa_ref[sl, :] = h[:, 0:HD]
        hsb_ref[sl, :] = h[:, HD:D]


def _y_chunk(aggp_ref, hsa_ref, hsb_ref, dinv_ref, b_ref, sl):
    dinv = dinv_ref[sl]
    ya = (aggp_ref[0, 0, sl, :] + aggp_ref[0, 1, sl, :]
          + hsa_ref[sl, :]) * dinv + b_ref[:, 0:HD]
    yb = (aggp_ref[1, 0, sl, :] + aggp_ref[1, 1, sl, :]
          + hsb_ref[sl, :]) * dinv + b_ref[:, HD:D]
    return jnp.concatenate([ya, yb], axis=1)


def _tc_mid_body(aggp_ref, hsa_ref, hsb_ref, dinv_ref, b_ref, g_ref, be_ref,
                 wn_ref, hsa_out, hsb_out):
    ssum = jnp.zeros((1, D), _f32)
    ssq = jnp.zeros((1, D), _f32)
    for i in range(N // TCC):
        sl = pl.ds(i * TCC, TCC)
        y = _y_chunk(aggp_ref, hsa_ref, hsb_ref, dinv_ref, b_ref, sl)
        ssum = ssum + jnp.sum(y, axis=0, keepdims=True)
        ssq = ssq + jnp.sum(y * y, axis=0, keepdims=True)
    m = ssum * (1.0 / N)
    v = ssq * (1.0 / N) - m * m
    scale = lax.rsqrt(v + 1e-5) * g_ref[...]
    shift = be_ref[...] - m * scale
    wn = wn_ref[...]
    for i in range(N // TCC):
        sl = pl.ds(i * TCC, TCC)
        y = _y_chunk(aggp_ref, hsa_ref, hsb_ref, dinv_ref, b_ref, sl)
        x = jnp.maximum(y * scale + shift, 0.0)
        h = _mm(x, wn) * dinv_ref[sl]
        hsa_out[sl, :] = h[:, 0:HD]
        hsb_out[sl, :] = h[:, HD:D]


def _tc_final_body(aggp_ref, hsa_ref, hsb_ref, dinv_ref, b_ref, fw_ref,
                   fb_ref, p_ref, q_ref):
    fwp = fw_ref[0:D]
    fwq = fw_ref[D:2 * D]
    fb = fb_ref[...]
    for i in range(N // TCC):
        sl = pl.ds(i * TCC, TCC)
        z = _y_chunk(aggp_ref, hsa_ref, hsb_ref, dinv_ref, b_ref, sl)
        p_ref[sl, :] = _mm(z, fwp)
        q_ref[sl, :] = _mm(z, fwq) + fb


def _tc_call(body, out_shapes):
    return pl.pallas_call(body, out_shape=out_shapes)


def kernel(node_feat, src, dst, neg, W1, b1, W2, b2, W3, b3,
           g1, be1, g2, be2, fc1_W, fc1_b, fc2_W, fc2_b):
    nd = jax.ShapeDtypeStruct((N, D), _f32)
    nh = jax.ShapeDtypeStruct((N, HD), _f32)
    n1 = jax.ShapeDtypeStruct((N, 1), _f32)
    row = lambda v: v.reshape(1, -1)

    degp = _deg_kernel(dst)
    hsa, hsb, dinv = _tc_call(_tc_prep_body, (nh, nh, n1))(
        node_feat, W1, degp)

    agg1 = _scatter_kernel(hsa, hsb, src, dst)
    hsa, hsb = _tc_call(_tc_mid_body, (nh, nh))(
        agg1, hsa, hsb, dinv, row(b1), row(g1), row(be1), W2)

    agg2 = _scatter_kernel(hsa, hsb, src, dst)
    hsa, hsb = _tc_call(_tc_mid_body, (nh, nh))(
        agg2, hsa, hsb, dinv, row(b2), row(g2), row(be2), W3)

    agg3 = _scatter_kernel(hsa, hsb, src, dst)
    P, Q = _tc_call(_tc_final_body, (nd, nd))(
        agg3, hsa, hsb, dinv, row(b3), fc1_W, row(fc1_b))

    wb = jnp.concatenate([fc2_W.reshape(-1),
                          jnp.broadcast_to(fc2_b.reshape(-1), (16,))])
    pos_out, neg_out = _decode_kernel(P, Q, src, dst, neg, wb)
    return pos_out, neg_out
